# baseline (speedup 1.0000x reference)
import jax
import jax.numpy as jnp
from jax.experimental import pallas as pl

D_MODEL = 128
MAX_LEN = 5000


def _rpe_kernel(x_ref, rev_ref, o_ref, *, block_i, seq_len):
    i0 = pl.program_id(0) * block_i
    for ii in range(block_i):
        start = (seq_len - 1) - i0 - ii
        o_ref[ii] = x_ref[:] + rev_ref[pl.ds(start, seq_len), :]


def kernel(inputs, rel_pos_encoding):
    _, seq_len, d = inputs.shape
    x = inputs[0]

    lo = MAX_LEN - (seq_len - 1)
    window = jax.lax.slice(rel_pos_encoding, (lo, 0), (lo + 2 * seq_len - 1, d))
    rev = jnp.flip(window, axis=0)
    rev = jnp.pad(rev, ((0, 1), (0, 0)))

    block_i = 8
    grid = seq_len // block_i

    out = pl.pallas_call(
        lambda x_ref, rev_ref, o_ref: _rpe_kernel(
            x_ref, rev_ref, o_ref, block_i=block_i, seq_len=seq_len
        ),
        grid=(grid,),
        in_specs=[
            pl.BlockSpec((seq_len, d), lambda g: (0, 0)),
            pl.BlockSpec((2 * seq_len, d), lambda g: (0, 0)),
        ],
        out_specs=pl.BlockSpec((block_i, seq_len, d), lambda g: (g, 0, 0)),
        out_shape=jax.ShapeDtypeStruct((seq_len, seq_len, d), inputs.dtype),
    )(x, rev)
    return out


# speedup vs baseline: 1.2015x; 1.2015x over previous
import jax
import jax.numpy as jnp
from jax.experimental import pallas as pl

D_MODEL = 128
MAX_LEN = 5000


def _rpe_kernel(x_ref, rev_ref, o_ref, *, block_i, seq_len):
    i0 = pl.program_id(0) * block_i
    for ii in range(block_i):
        start = (seq_len - 1) - i0 - ii
        o_ref[ii] = x_ref[:] + rev_ref[pl.ds(start, seq_len), :]


def kernel(inputs, rel_pos_encoding):
    _, seq_len, d = inputs.shape
    x = inputs[0]

    lo = MAX_LEN - (seq_len - 1)
    window = jax.lax.slice(rel_pos_encoding, (lo, 0), (lo + 2 * seq_len - 1, d))
    rev = jnp.flip(window, axis=0)
    rev = jnp.pad(rev, ((0, 1), (0, 0)))

    block_i = 32
    grid = seq_len // block_i

    out = pl.pallas_call(
        lambda x_ref, rev_ref, o_ref: _rpe_kernel(
            x_ref, rev_ref, o_ref, block_i=block_i, seq_len=seq_len
        ),
        grid=(grid,),
        in_specs=[
            pl.BlockSpec((seq_len, d), lambda g: (0, 0)),
            pl.BlockSpec((2 * seq_len, d), lambda g: (0, 0)),
        ],
        out_specs=pl.BlockSpec((block_i, seq_len, d), lambda g: (g, 0, 0)),
        out_shape=jax.ShapeDtypeStruct((seq_len, seq_len, d), inputs.dtype),
    )(x, rev)
    return out


# speedup vs baseline: 1.2222x; 1.0173x over previous
import jax
import jax.numpy as jnp
from jax.experimental import pallas as pl

D_MODEL = 128
MAX_LEN = 5000


def _rpe_kernel(x_ref, rev_ref, o_ref, *, block_i, seq_len):
    i0 = pl.program_id(0) * block_i
    for ii in range(block_i):
        start = (seq_len - 1) - i0 - ii
        o_ref[ii] = x_ref[:] + rev_ref[pl.ds(start, seq_len), :]


def kernel(inputs, rel_pos_encoding):
    _, seq_len, d = inputs.shape
    x = inputs[0]

    lo = MAX_LEN - (seq_len - 1)
    window = jax.lax.slice(rel_pos_encoding, (lo, 0), (lo + 2 * seq_len - 1, d))
    rev = jnp.flip(window, axis=0)
    rev = jnp.pad(rev, ((0, 1), (0, 0)))

    block_i = 16
    grid = seq_len // block_i

    out = pl.pallas_call(
        lambda x_ref, rev_ref, o_ref: _rpe_kernel(
            x_ref, rev_ref, o_ref, block_i=block_i, seq_len=seq_len
        ),
        grid=(grid,),
        in_specs=[
            pl.BlockSpec((seq_len, d), lambda g: (0, 0)),
            pl.BlockSpec((2 * seq_len, d), lambda g: (0, 0)),
        ],
        out_specs=pl.BlockSpec((block_i, seq_len, d), lambda g: (g, 0, 0)),
        out_shape=jax.ShapeDtypeStruct((seq_len, seq_len, d), inputs.dtype),
    )(x, rev)
    return out
